# Initial kernel scaffold; baseline (speedup 1.0000x reference)
import jax
import jax.numpy as jnp
from jax.experimental import pallas as pl


def kernel(x, table, W):
    raise NotImplementedError("write your pallas kernel here")


# speedup vs baseline: 14.2447x; 14.2447x over previous
import functools

import jax
import jax.numpy as jnp
from jax import lax
from jax.experimental import pallas as pl
from jax.experimental.pallas import tpu as pltpu
from jax.experimental.pallas import tpu_sc as plsc

BATCH = 16384
N_GROUPS = 26
INNER = 128
OUT_DIM = 128
N_IDX = BATCH * N_GROUPS
NW = 32
PER_W = N_IDX // NW
CHUNK = 128
N_CHUNK = PER_W // CHUNK


def _gather_body(table_hbm, idx_hbm, out_hbm, idx_v, rows_v, sem_g):
    c = lax.axis_index("c")
    s = lax.axis_index("s")
    wid = s * 2 + c
    pltpu.sync_copy(idx_hbm.at[wid], idx_v)

    def body(g, carry):
        base = wid * PER_W + g * CHUNK
        pltpu.async_copy(table_hbm.at[idx_v.at[g]], rows_v, sem_g).wait()
        pltpu.sync_copy(rows_v, out_hbm.at[pl.ds(base, CHUNK)])
        return carry

    lax.fori_loop(0, N_CHUNK, body, 0)


def _sc_gather(table, idx3):
    mesh = plsc.VectorSubcoreMesh(core_axis_name="c", subcore_axis_name="s")
    kern = functools.partial(
        pl.kernel,
        mesh=mesh,
        out_type=jax.ShapeDtypeStruct((N_IDX, INNER), jnp.float32),
        scratch_types=[
            pltpu.VMEM((N_CHUNK, CHUNK), jnp.int32),
            pltpu.VMEM((CHUNK, INNER), jnp.float32),
            pltpu.SemaphoreType.DMA,
        ],
    )(_gather_body)
    return kern(table, idx3)


def _mm_body(flat_ref, wt_ref, out_ref):
    out_ref[...] = jnp.dot(flat_ref[...], wt_ref[...],
                           preferred_element_type=jnp.float32)


def _tc_matmul(flat, wt):
    bm = 1024
    return pl.pallas_call(
        _mm_body,
        grid=(BATCH // bm,),
        in_specs=[
            pl.BlockSpec((bm, N_GROUPS * INNER), lambda i: (i, 0)),
            pl.BlockSpec((N_GROUPS * INNER, OUT_DIM), lambda i: (0, 0)),
        ],
        out_specs=pl.BlockSpec((bm, OUT_DIM), lambda i: (i, 0)),
        out_shape=jax.ShapeDtypeStruct((BATCH, OUT_DIM), jnp.float32),
    )(flat, wt)


def kernel(x, table, W):
    idx3 = x.reshape(NW, N_CHUNK, CHUNK)
    rows = _sc_gather(table, idx3)
    flat = rows.reshape(BATCH, N_GROUPS * INNER)
    return _tc_matmul(flat, W.T)


# speedup vs baseline: 16.5964x; 1.1651x over previous
import functools

import jax
import jax.numpy as jnp
from jax import lax
from jax.experimental import pallas as pl
from jax.experimental.pallas import tpu as pltpu
from jax.experimental.pallas import tpu_sc as plsc

BATCH = 16384
N_GROUPS = 26
INNER = 128
OUT_DIM = 128
N_IDX = BATCH * N_GROUPS
NW = 32
PER_W = N_IDX // NW
CHUNK = 128
N_CHUNK = PER_W // CHUNK
RING = 4
LOOK = 2
NG = N_CHUNK // RING


def _gather_body(table_hbm, idx_hbm, out_hbm, idx_v,
                 b0, b1, b2, b3, g0, g1, g2, g3, o0, o1, o2, o3):
    bufs = [b0, b1, b2, b3]
    gsem = [g0, g1, g2, g3]
    osem = [o0, o1, o2, o3]
    wid = lax.axis_index("s") * 2 + lax.axis_index("c")
    obase = wid * PER_W
    pltpu.sync_copy(idx_hbm.at[wid], idx_v)

    def start_gather(g, b):
        pltpu.async_copy(table_hbm.at[idx_v.at[g]], bufs[b], gsem[b])

    def wait_gather(b):
        pltpu.make_async_copy(table_hbm.at[pl.ds(0, CHUNK)], bufs[b],
                              gsem[b]).wait()

    def start_out(g, b):
        pltpu.async_copy(bufs[b], out_hbm.at[pl.ds(obase + g * CHUNK, CHUNK)],
                         osem[b])

    def wait_out(b):
        pltpu.make_async_copy(bufs[b], out_hbm.at[pl.ds(0, CHUNK)],
                              osem[b]).wait()

    for g in range(LOOK):
        start_gather(g, g)

    for b in range(RING):
        wait_gather(b)
        start_out(b, b)
        la, j = b + LOOK, (b + LOOK) % RING
        if la >= RING:
            wait_out(j)
        start_gather(la, j)

    def group(grp, carry):
        g_base = grp * RING
        for b in range(RING):
            wait_gather(b)
            start_out(g_base + b, b)
            j = (b + LOOK) % RING
            wait_out(j)
            start_gather(g_base + b + LOOK, j)
        return carry

    lax.fori_loop(1, NG - 1, group, 0)

    g_base = (NG - 1) * RING
    for b in range(RING):
        wait_gather(b)
        start_out(g_base + b, b)
        la = g_base + b + LOOK
        if la < N_CHUNK:
            j = (b + LOOK) % RING
            wait_out(j)
            start_gather(la, j)

    for b in range(RING):
        wait_out(b)


def _sc_gather(table, idx3):
    mesh = plsc.VectorSubcoreMesh(core_axis_name="c", subcore_axis_name="s")
    kern = functools.partial(
        pl.kernel,
        mesh=mesh,
        out_type=jax.ShapeDtypeStruct((N_IDX, INNER), jnp.float32),
        scratch_types=[
            pltpu.VMEM((N_CHUNK, CHUNK), jnp.int32),
            pltpu.VMEM((CHUNK, INNER), jnp.float32),
            pltpu.VMEM((CHUNK, INNER), jnp.float32),
            pltpu.VMEM((CHUNK, INNER), jnp.float32),
            pltpu.VMEM((CHUNK, INNER), jnp.float32),
            pltpu.SemaphoreType.DMA,
            pltpu.SemaphoreType.DMA,
            pltpu.SemaphoreType.DMA,
            pltpu.SemaphoreType.DMA,
            pltpu.SemaphoreType.DMA,
            pltpu.SemaphoreType.DMA,
            pltpu.SemaphoreType.DMA,
            pltpu.SemaphoreType.DMA,
        ],
    )(_gather_body)
    return kern(table, idx3)


def _mm_body(flat_ref, wt_ref, out_ref):
    out_ref[...] = jnp.dot(flat_ref[...], wt_ref[...],
                           preferred_element_type=jnp.float32)


def _tc_matmul(flat, wt):
    bm = 1024
    return pl.pallas_call(
        _mm_body,
        grid=(BATCH // bm,),
        in_specs=[
            pl.BlockSpec((bm, N_GROUPS * INNER), lambda i: (i, 0)),
            pl.BlockSpec((N_GROUPS * INNER, OUT_DIM), lambda i: (0, 0)),
        ],
        out_specs=pl.BlockSpec((bm, OUT_DIM), lambda i: (i, 0)),
        out_shape=jax.ShapeDtypeStruct((BATCH, OUT_DIM), jnp.float32),
    )(flat, wt)


def kernel(x, table, W):
    idx3 = x.reshape(NW, N_CHUNK, CHUNK)
    rows = _sc_gather(table, idx3)
    flat = rows.reshape(BATCH, N_GROUPS * INNER)
    return _tc_matmul(flat, W.T)
